# baseline (speedup 1.0000x reference)
import jax
import jax.numpy as jnp
from jax.experimental import pallas as pl


def _gather_mm(x_ref, m_ref, o_ref):
    o_ref[...] = jnp.dot(x_ref[...], m_ref[...],
                         preferred_element_type=jnp.float32)


def kernel(x, inds):
    n_rows, n_cols = x.shape
    k = inds.shape[0]
    m = (inds[None, :] == jnp.arange(n_cols, dtype=inds.dtype)[:, None])
    m = m.astype(x.dtype)

    block_rows = 512
    grid = (n_rows // block_rows,)
    return pl.pallas_call(
        _gather_mm,
        grid=grid,
        in_specs=[
            pl.BlockSpec((block_rows, n_cols), lambda i: (i, 0)),
            pl.BlockSpec((n_cols, k), lambda i: (0, 0)),
        ],
        out_specs=pl.BlockSpec((block_rows, k), lambda i: (i, 0)),
        out_shape=jax.ShapeDtypeStruct((n_rows, k), x.dtype),
    )(x, m)


# speedup vs baseline: 1.4600x; 1.4600x over previous
import jax
import jax.numpy as jnp
from jax.experimental import pallas as pl


def _gather_mm(x_ref, m_ref, o_ref):
    o_ref[...] = jnp.dot(x_ref[...], m_ref[...],
                         preferred_element_type=jnp.float32)


def kernel(x, inds):
    n_rows, n_cols = x.shape
    k = inds.shape[0]
    m = (inds[None, :] == jnp.arange(n_cols, dtype=inds.dtype)[:, None])
    m = m.astype(x.dtype)

    block_rows = 2048
    grid = (n_rows // block_rows,)
    return pl.pallas_call(
        _gather_mm,
        grid=grid,
        in_specs=[
            pl.BlockSpec((block_rows, n_cols), lambda i: (i, 0)),
            pl.BlockSpec((n_cols, k), lambda i: (0, 0)),
        ],
        out_specs=pl.BlockSpec((block_rows, k), lambda i: (i, 0)),
        out_shape=jax.ShapeDtypeStruct((n_rows, k), x.dtype),
    )(x, m)
